# baseline (speedup 1.0000x reference)
import functools
import jax
import jax.numpy as jnp
from jax import lax
from jax.experimental import pallas as pl
from jax.experimental.pallas import tpu as pltpu
from jax.experimental.pallas import tpu_sc as plsc

_info = plsc.get_sparse_core_info()
_NC, _NS = _info.num_cores, _info.num_subcores
_NW = _NC * _NS

_CHUNK = 8
_NBUF = 4


def kernel(x, W_pos):
    seq_len, d_model = W_pos.shape
    rows_per_w = seq_len // _NW
    nchunk = rows_per_w // _CHUNK
    ngroup = nchunk // _NBUF
    mesh = plsc.VectorSubcoreMesh(core_axis_name="c", subcore_axis_name="s")

    @functools.partial(
        pl.kernel,
        mesh=mesh,
        out_type=jax.ShapeDtypeStruct((seq_len, d_model), W_pos.dtype),
        scratch_types=[
            pltpu.VMEM((_NBUF, _CHUNK, d_model), W_pos.dtype),
            pltpu.SemaphoreType.DMA((_NBUF,)),
            pltpu.SemaphoreType.DMA((_NBUF,)),
        ],
    )
    def k(w_hbm, out_hbm, buf, insem, outsem):
        wid = lax.axis_index("s") * _NC + lax.axis_index("c")
        base = wid * rows_per_w

        def in_copy(c, b):
            return pltpu.make_async_copy(
                w_hbm.at[pl.ds(base + c * _CHUNK, _CHUNK)],
                buf.at[b],
                insem.at[b],
            )

        def out_copy(c, b):
            return pltpu.make_async_copy(
                buf.at[b],
                out_hbm.at[pl.ds(base + c * _CHUNK, _CHUNK)],
                outsem.at[b],
            )

        def body(g, _):
            for b in range(_NBUF):
                c = g * _NBUF + b

                @pl.when(g > 0)
                def _():
                    out_copy(c - _NBUF, b).wait()

                in_copy(c, b).start()
            for b in range(_NBUF):
                c = g * _NBUF + b
                in_copy(c, b).wait()
                out_copy(c, b).start()
            return 0

        lax.fori_loop(0, ngroup, body, 0)
        for b in range(_NBUF):
            c = (ngroup - 1) * _NBUF + b
            out_copy(c, b).wait()

    return k(W_pos)


# speedup vs baseline: 1.0973x; 1.0973x over previous
import functools
import jax
import jax.numpy as jnp
from jax import lax
from jax.experimental import pallas as pl
from jax.experimental.pallas import tpu as pltpu
from jax.experimental.pallas import tpu_sc as plsc

_info = plsc.get_sparse_core_info()
_NC, _NS = _info.num_cores, _info.num_subcores
_NW = _NC * _NS

_CHUNK = 16
_NBUF = 2


def kernel(x, W_pos):
    seq_len, d_model = W_pos.shape
    rows_per_w = seq_len // _NW
    nchunk = rows_per_w // _CHUNK
    ngroup = nchunk // _NBUF
    mesh = plsc.VectorSubcoreMesh(core_axis_name="c", subcore_axis_name="s")

    @functools.partial(
        pl.kernel,
        mesh=mesh,
        out_type=jax.ShapeDtypeStruct((seq_len, d_model), W_pos.dtype),
        scratch_types=[
            pltpu.MemorySpace.VMEM_SHARED((_NS, _NBUF, _CHUNK, d_model), W_pos.dtype),
            pltpu.SemaphoreType.DMA((_NBUF,)),
            pltpu.SemaphoreType.DMA((_NBUF,)),
        ],
    )
    def k(w_hbm, out_hbm, buf, insem, outsem):
        sid = lax.axis_index("s")
        wid = sid * _NC + lax.axis_index("c")
        base = wid * rows_per_w

        def in_copy(c, b):
            return pltpu.make_async_copy(
                w_hbm.at[pl.ds(base + c * _CHUNK, _CHUNK)],
                buf.at[sid, b],
                insem.at[b],
            )

        def out_copy(c, b):
            return pltpu.make_async_copy(
                buf.at[sid, b],
                out_hbm.at[pl.ds(base + c * _CHUNK, _CHUNK)],
                outsem.at[b],
            )

        def body(g, _):
            for b in range(_NBUF):
                c = g * _NBUF + b

                @pl.when(g > 0)
                def _():
                    out_copy(c - _NBUF, b).wait()

                in_copy(c, b).start()
            for b in range(_NBUF):
                c = g * _NBUF + b
                in_copy(c, b).wait()
                out_copy(c, b).start()
            return 0

        lax.fori_loop(0, ngroup, body, 0)
        for b in range(_NBUF):
            c = (ngroup - 1) * _NBUF + b
            out_copy(c, b).wait()

    return k(W_pos)
